# initial kernel scaffold (unmeasured)
import jax
import jax.numpy as jnp
from jax import lax
from jax.experimental import pallas as pl
from jax.experimental.pallas import tpu as pltpu

N_DEV = 4


def kernel(x, w_mat):
    m, k_per = x.shape
    _, n = w_mat.shape
    mc = m // N_DEV

    def body(x_ref, w_ref, out_ref, acc_ref, recv_ref, send_sem, recv_sem,
             credit_sem):
        d = lax.axis_index("i")
        left = lax.rem(d + N_DEV - 1, N_DEV)
        right = lax.rem(d + 1, N_DEV)

        barrier_sem = pltpu.get_barrier_semaphore()
        for nbr in (left, right):
            pl.semaphore_signal(
                barrier_sem, inc=1,
                device_id=(nbr,), device_id_type=pl.DeviceIdType.MESH,
            )
        pl.semaphore_wait(barrier_sem, 2)

        def partial(c):
            rows = x_ref[pl.ds(c * mc, mc), :]
            return jnp.dot(rows, w_ref[:, :],
                           preferred_element_type=jnp.float32)

        prev = None
        for h in range(2 * (N_DEV - 1)):
            if prev is not None:
                prev.wait_send()
            if h < N_DEV - 1:
                c = lax.rem(d - h + 2 * N_DEV, N_DEV)
                if h == 0:
                    acc_ref[...] = partial(c)
                else:
                    acc_ref[...] = recv_ref[...] + partial(c)
            else:
                t = h - (N_DEV - 1)
                c = lax.rem(d + 1 - t + 2 * N_DEV, N_DEV)
                if t == 0:
                    acc_ref[...] = recv_ref[...] + partial(c)
                else:
                    acc_ref[...] = recv_ref[...]
                out_ref[pl.ds(c * mc, mc), :] = jnp.maximum(acc_ref[...], 0.0)
            if h > 0:
                pl.semaphore_signal(
                    credit_sem, inc=1,
                    device_id=(left,), device_id_type=pl.DeviceIdType.MESH,
                )
                pl.semaphore_wait(credit_sem, 1)
            rdma = pltpu.make_async_remote_copy(
                src_ref=acc_ref,
                dst_ref=recv_ref,
                send_sem=send_sem,
                recv_sem=recv_sem,
                device_id=(right,),
                device_id_type=pl.DeviceIdType.MESH,
            )
            rdma.start()
            rdma.wait_recv()
            prev = rdma

        c_last = lax.rem(d - 2 + 2 * N_DEV, N_DEV)
        out_ref[pl.ds(c_last * mc, mc), :] = jnp.maximum(recv_ref[...], 0.0)
        prev.wait_send()

    return pl.pallas_call(
        body,
        out_shape=jax.ShapeDtypeStruct((m, n), jnp.float32),
        in_specs=[
            pl.BlockSpec(memory_space=pltpu.VMEM),
            pl.BlockSpec(memory_space=pltpu.VMEM),
        ],
        out_specs=pl.BlockSpec(memory_space=pltpu.VMEM),
        scratch_shapes=[
            pltpu.VMEM((mc, n), jnp.float32),
            pltpu.VMEM((mc, n), jnp.float32),
            pltpu.SemaphoreType.DMA,
            pltpu.SemaphoreType.DMA,
            pltpu.SemaphoreType.REGULAR,
        ],
        compiler_params=pltpu.CompilerParams(collective_id=0),
    )(x, w_mat)


# baseline (device time: 612635 ns/iter reference)
import jax
import jax.numpy as jnp
from jax import lax
from jax.experimental import pallas as pl
from jax.experimental.pallas import tpu as pltpu

N_DEV = 4


def kernel(x, w_mat):
    m, k_per = x.shape
    _, n = w_mat.shape
    mc = m // N_DEV

    def body(x_ref, w_ref, out_ref, acc_ref, recv_ref, send_sem, recv_sem,
             copy_sem, credit_sem):
        d = lax.axis_index("i")
        left = lax.rem(d + N_DEV - 1, N_DEV)
        right = lax.rem(d + 1, N_DEV)

        barrier_sem = pltpu.get_barrier_semaphore()
        for nbr in (left, right):
            pl.semaphore_signal(
                barrier_sem, inc=1,
                device_id=(nbr,), device_id_type=pl.DeviceIdType.MESH,
            )
        pl.semaphore_wait(barrier_sem, 2)

        def partial(c):
            rows = x_ref[pl.ds(c * mc, mc), :]
            return jnp.dot(rows, w_ref[:, :],
                           preferred_element_type=jnp.float32)

        prev_rdma = None
        pending_copy = None
        for h in range(2 * (N_DEV - 1)):
            if prev_rdma is not None:
                prev_rdma.wait_send()
            if pending_copy is not None:
                pending_copy.wait()
                pending_copy = None
            if h < N_DEV - 1:
                c = lax.rem(d - h + 2 * N_DEV, N_DEV)
                if h == 0:
                    acc_ref[...] = partial(c)
                else:
                    acc_ref[...] = recv_ref[...] + partial(c)
            else:
                t = h - (N_DEV - 1)
                c = lax.rem(d + 1 - t + 2 * N_DEV, N_DEV)
                if t == 0:
                    acc_ref[...] = jnp.maximum(
                        recv_ref[...] + partial(c), 0.0)
                else:
                    acc_ref[...] = recv_ref[...]
                cp = pltpu.make_async_copy(
                    acc_ref, out_ref.at[pl.ds(c * mc, mc), :], copy_sem)
                cp.start()
                pending_copy = cp
            if h > 0:
                pl.semaphore_signal(
                    credit_sem, inc=1,
                    device_id=(left,), device_id_type=pl.DeviceIdType.MESH,
                )
                pl.semaphore_wait(credit_sem, 1)
            rdma = pltpu.make_async_remote_copy(
                src_ref=acc_ref,
                dst_ref=recv_ref,
                send_sem=send_sem,
                recv_sem=recv_sem,
                device_id=(right,),
                device_id_type=pl.DeviceIdType.MESH,
            )
            rdma.start()
            rdma.wait_recv()
            prev_rdma = rdma

        pending_copy.wait()
        c_last = lax.rem(d - 2 + 2 * N_DEV, N_DEV)
        cp = pltpu.make_async_copy(
            recv_ref, out_ref.at[pl.ds(c_last * mc, mc), :], copy_sem)
        cp.start()
        cp.wait()
        prev_rdma.wait_send()

    return pl.pallas_call(
        body,
        out_shape=jax.ShapeDtypeStruct((m, n), jnp.float32),
        in_specs=[
            pl.BlockSpec(memory_space=pltpu.VMEM),
            pl.BlockSpec(memory_space=pltpu.VMEM),
        ],
        out_specs=pl.BlockSpec(memory_space=pl.ANY),
        scratch_shapes=[
            pltpu.VMEM((mc, n), jnp.float32),
            pltpu.VMEM((mc, n), jnp.float32),
            pltpu.SemaphoreType.DMA,
            pltpu.SemaphoreType.DMA,
            pltpu.SemaphoreType.DMA,
            pltpu.SemaphoreType.REGULAR,
        ],
        compiler_params=pltpu.CompilerParams(collective_id=0),
    )(x, w_mat)


# device time: 330637 ns/iter; 1.8529x vs baseline; 1.8529x over previous
import jax
import jax.numpy as jnp
from jax import lax
from jax.experimental import pallas as pl
from jax.experimental.pallas import tpu as pltpu

N_DEV = 4
N_HOPS = 2 * (N_DEV - 1)


def kernel(x, w_mat):
    m, k_per = x.shape
    _, n = w_mat.shape
    mc = m // N_DEV
    n2 = n // 2

    def body(x_ref, w_ref, out_ref,
             accA, accB, recvA, recvB, pnxA, pnxB,
             sendA_sem, sendB_sem, recvA_sem, recvB_sem,
             copyA_sem, copyB_sem, creditA_sem, creditB_sem):
        d = lax.axis_index("i")
        left = lax.rem(d + N_DEV - 1, N_DEV)
        right = lax.rem(d + 1, N_DEV)

        barrier_sem = pltpu.get_barrier_semaphore()
        for nbr in (left, right):
            pl.semaphore_signal(
                barrier_sem, inc=1,
                device_id=(nbr,), device_id_type=pl.DeviceIdType.MESH,
            )
        pl.semaphore_wait(barrier_sem, 2)

        def pA(c):
            return jnp.dot(x_ref[pl.ds(c * mc, mc), :], w_ref[:, :n2],
                           preferred_element_type=jnp.float32)

        def pB(c):
            return jnp.dot(x_ref[pl.ds(c * mc, mc), :], w_ref[:, n2:],
                           preferred_element_type=jnp.float32)

        def chunk_A(h):
            if h < N_DEV - 1:
                return lax.rem(d - h + 2 * N_DEV, N_DEV)
            return lax.rem(d + 1 - (h - (N_DEV - 1)) + 2 * N_DEV, N_DEV)

        def chunk_B(h):
            if h < N_DEV - 1:
                return lax.rem(d + h, N_DEV)
            return lax.rem(d - 1 + (h - (N_DEV - 1)) + 2 * N_DEV, N_DEV)

        prevA = prevB = None
        pendA = pendB = None
        for h in range(N_HOPS):
            if prevA is not None:
                prevA.wait_send()
                prevB.wait_send()
            if pendA is not None:
                pendA.wait()
                pendB.wait()
                pendA = pendB = None
            cA, cB = chunk_A(h), chunk_B(h)
            if h == 0:
                accA[...] = pA(cA)
                accB[...] = pB(cB)
            elif h < N_DEV - 1:
                accA[...] = recvA[...] + pnxA[...]
                accB[...] = recvB[...] + pnxB[...]
            elif h == N_DEV - 1:
                accA[...] = jnp.maximum(recvA[...] + pnxA[...], 0.0)
                accB[...] = jnp.maximum(recvB[...] + pnxB[...], 0.0)
            else:
                accA[...] = recvA[...]
                accB[...] = recvB[...]
            if h >= N_DEV - 1:
                pendA = pltpu.make_async_copy(
                    accA, out_ref.at[pl.ds(cA * mc, mc), :n2], copyA_sem)
                pendB = pltpu.make_async_copy(
                    accB, out_ref.at[pl.ds(cB * mc, mc), n2:], copyB_sem)
                pendA.start()
                pendB.start()
            if h > 0:
                pl.semaphore_signal(
                    creditA_sem, inc=1,
                    device_id=(left,), device_id_type=pl.DeviceIdType.MESH)
                pl.semaphore_signal(
                    creditB_sem, inc=1,
                    device_id=(right,), device_id_type=pl.DeviceIdType.MESH)
                pl.semaphore_wait(creditA_sem, 1)
                pl.semaphore_wait(creditB_sem, 1)
            rdmaA = pltpu.make_async_remote_copy(
                src_ref=accA, dst_ref=recvA,
                send_sem=sendA_sem, recv_sem=recvA_sem,
                device_id=(right,), device_id_type=pl.DeviceIdType.MESH)
            rdmaB = pltpu.make_async_remote_copy(
                src_ref=accB, dst_ref=recvB,
                send_sem=sendB_sem, recv_sem=recvB_sem,
                device_id=(left,), device_id_type=pl.DeviceIdType.MESH)
            rdmaA.start()
            rdmaB.start()
            if h < N_DEV - 1:
                pnxA[...] = pA(chunk_A(h + 1))
                pnxB[...] = pB(chunk_B(h + 1))
            rdmaA.wait_recv()
            rdmaB.wait_recv()
            prevA, prevB = rdmaA, rdmaB

        pendA.wait()
        pendB.wait()
        c_last = lax.rem(d + 2, N_DEV)
        cpA = pltpu.make_async_copy(
            recvA, out_ref.at[pl.ds(c_last * mc, mc), :n2], copyA_sem)
        cpB = pltpu.make_async_copy(
            recvB, out_ref.at[pl.ds(c_last * mc, mc), n2:], copyB_sem)
        cpA.start()
        cpB.start()
        cpA.wait()
        cpB.wait()
        prevA.wait_send()
        prevB.wait_send()

    return pl.pallas_call(
        body,
        out_shape=jax.ShapeDtypeStruct((m, n), jnp.float32),
        in_specs=[
            pl.BlockSpec(memory_space=pltpu.VMEM),
            pl.BlockSpec(memory_space=pltpu.VMEM),
        ],
        out_specs=pl.BlockSpec(memory_space=pl.ANY),
        scratch_shapes=[
            pltpu.VMEM((mc, n2), jnp.float32),
            pltpu.VMEM((mc, n2), jnp.float32),
            pltpu.VMEM((mc, n2), jnp.float32),
            pltpu.VMEM((mc, n2), jnp.float32),
            pltpu.VMEM((mc, n2), jnp.float32),
            pltpu.VMEM((mc, n2), jnp.float32),
            pltpu.SemaphoreType.DMA,
            pltpu.SemaphoreType.DMA,
            pltpu.SemaphoreType.DMA,
            pltpu.SemaphoreType.DMA,
            pltpu.SemaphoreType.DMA,
            pltpu.SemaphoreType.DMA,
            pltpu.SemaphoreType.REGULAR,
            pltpu.SemaphoreType.REGULAR,
        ],
        compiler_params=pltpu.CompilerParams(
            collective_id=0,
            vmem_limit_bytes=40 * 1024 * 1024,
        ),
    )(x, w_mat)


# device time: 311390 ns/iter; 1.9674x vs baseline; 1.0618x over previous
import jax
import jax.numpy as jnp
from jax import lax
from jax.experimental import pallas as pl
from jax.experimental.pallas import tpu as pltpu

N_DEV = 4
N_HOPS = 2 * (N_DEV - 1)
N_SUB = 2


def kernel(x, w_mat):
    m, k_per = x.shape
    _, n = w_mat.shape
    mc = m // N_DEV
    ms = mc // N_SUB
    n2 = n // 2

    def body(x_ref, w_ref, out_ref,
             accA, accB, recvA, recvB, pnxA, pnxB,
             sendA_sems, sendB_sems, recvA_sems, recvB_sems,
             copyA_sems, copyB_sems, creditA_sem, creditB_sem):
        d = lax.axis_index("i")
        left = lax.rem(d + N_DEV - 1, N_DEV)
        right = lax.rem(d + 1, N_DEV)

        barrier_sem = pltpu.get_barrier_semaphore()
        for nbr in (left, right):
            pl.semaphore_signal(
                barrier_sem, inc=1,
                device_id=(nbr,), device_id_type=pl.DeviceIdType.MESH,
            )
        pl.semaphore_wait(barrier_sem, 2)

        def pA(c, s):
            return jnp.dot(x_ref[pl.ds(c * mc + s * ms, ms), :],
                           w_ref[:, :n2],
                           preferred_element_type=jnp.float32)

        def pB(c, s):
            return jnp.dot(x_ref[pl.ds(c * mc + s * ms, ms), :],
                           w_ref[:, n2:],
                           preferred_element_type=jnp.float32)

        def chunk_A(h):
            if h < N_DEV - 1:
                return lax.rem(d - h + 2 * N_DEV, N_DEV)
            return lax.rem(d + 1 - (h - (N_DEV - 1)) + 2 * N_DEV, N_DEV)

        def chunk_B(h):
            if h < N_DEV - 1:
                return lax.rem(d + h, N_DEV)
            return lax.rem(d - 1 + (h - (N_DEV - 1)) + 2 * N_DEV, N_DEV)

        descA = {}
        descB = {}
        pendA = {}
        pendB = {}

        for h in range(N_HOPS):
            cA, cB = chunk_A(h), chunk_B(h)
            for s in range(N_SUB):
                sA, sB = accA.at[pl.ds(s * ms, ms), :], accB.at[pl.ds(s * ms, ms), :]
                rA, rB = recvA.at[pl.ds(s * ms, ms), :], recvB.at[pl.ds(s * ms, ms), :]
                if h > 0:
                    descA[(h - 1, s)].wait_send()
                    descB[(h - 1, s)].wait_send()
                    descA[(h - 1, s)].wait_recv()
                    descB[(h - 1, s)].wait_recv()
                if s in pendA:
                    pendA.pop(s).wait()
                    pendB.pop(s).wait()
                if h == 0:
                    sA[...] = pA(cA, s)
                    sB[...] = pB(cB, s)
                elif h < N_DEV - 1:
                    sA[...] = rA[...] + pnxA[pl.ds(s * ms, ms), :]
                    sB[...] = rB[...] + pnxB[pl.ds(s * ms, ms), :]
                elif h == N_DEV - 1:
                    sA[...] = jnp.maximum(
                        rA[...] + pnxA[pl.ds(s * ms, ms), :], 0.0)
                    sB[...] = jnp.maximum(
                        rB[...] + pnxB[pl.ds(s * ms, ms), :], 0.0)
                else:
                    sA[...] = rA[...]
                    sB[...] = rB[...]
                if h >= N_DEV - 1:
                    cpA = pltpu.make_async_copy(
                        sA, out_ref.at[pl.ds(cA * mc + s * ms, ms), :n2],
                        copyA_sems.at[s])
                    cpB = pltpu.make_async_copy(
                        sB, out_ref.at[pl.ds(cB * mc + s * ms, ms), n2:],
                        copyB_sems.at[s])
                    cpA.start()
                    cpB.start()
                    pendA[s], pendB[s] = cpA, cpB
                if h > 0:
                    pl.semaphore_signal(
                        creditA_sem, inc=1,
                        device_id=(left,), device_id_type=pl.DeviceIdType.MESH)
                    pl.semaphore_signal(
                        creditB_sem, inc=1,
                        device_id=(right,), device_id_type=pl.DeviceIdType.MESH)
                    pl.semaphore_wait(creditA_sem, 1)
                    pl.semaphore_wait(creditB_sem, 1)
                rdmaA = pltpu.make_async_remote_copy(
                    src_ref=sA, dst_ref=rA,
                    send_sem=sendA_sems.at[s], recv_sem=recvA_sems.at[s],
                    device_id=(right,), device_id_type=pl.DeviceIdType.MESH)
                rdmaB = pltpu.make_async_remote_copy(
                    src_ref=sB, dst_ref=rB,
                    send_sem=sendB_sems.at[s], recv_sem=recvB_sems.at[s],
                    device_id=(left,), device_id_type=pl.DeviceIdType.MESH)
                rdmaA.start()
                rdmaB.start()
                descA[(h, s)], descB[(h, s)] = rdmaA, rdmaB
            if h < N_DEV - 1:
                for s in range(N_SUB):
                    pnxA[pl.ds(s * ms, ms), :] = pA(chunk_A(h + 1), s)
                    pnxB[pl.ds(s * ms, ms), :] = pB(chunk_B(h + 1), s)

        c_last = lax.rem(d + 2, N_DEV)
        h_last = N_HOPS - 1
        for s in range(N_SUB):
            descA[(h_last, s)].wait_recv()
            descB[(h_last, s)].wait_recv()
            pendA.pop(s).wait()
            pendB.pop(s).wait()
            cpA = pltpu.make_async_copy(
                recvA.at[pl.ds(s * ms, ms), :],
                out_ref.at[pl.ds(c_last * mc + s * ms, ms), :n2],
                copyA_sems.at[s])
            cpB = pltpu.make_async_copy(
                recvB.at[pl.ds(s * ms, ms), :],
                out_ref.at[pl.ds(c_last * mc + s * ms, ms), n2:],
                copyB_sems.at[s])
            cpA.start()
            cpB.start()
            pendA[s], pendB[s] = cpA, cpB
        for s in range(N_SUB):
            pendA.pop(s).wait()
            pendB.pop(s).wait()
            descA[(h_last, s)].wait_send()
            descB[(h_last, s)].wait_send()

    return pl.pallas_call(
        body,
        out_shape=jax.ShapeDtypeStruct((m, n), jnp.float32),
        in_specs=[
            pl.BlockSpec(memory_space=pltpu.VMEM),
            pl.BlockSpec(memory_space=pltpu.VMEM),
        ],
        out_specs=pl.BlockSpec(memory_space=pl.ANY),
        scratch_shapes=[
            pltpu.VMEM((mc, n2), jnp.float32),
            pltpu.VMEM((mc, n2), jnp.float32),
            pltpu.VMEM((mc, n2), jnp.float32),
            pltpu.VMEM((mc, n2), jnp.float32),
            pltpu.VMEM((mc, n2), jnp.float32),
            pltpu.VMEM((mc, n2), jnp.float32),
            pltpu.SemaphoreType.DMA((N_SUB,)),
            pltpu.SemaphoreType.DMA((N_SUB,)),
            pltpu.SemaphoreType.DMA((N_SUB,)),
            pltpu.SemaphoreType.DMA((N_SUB,)),
            pltpu.SemaphoreType.DMA((N_SUB,)),
            pltpu.SemaphoreType.DMA((N_SUB,)),
            pltpu.SemaphoreType.REGULAR,
            pltpu.SemaphoreType.REGULAR,
        ],
        compiler_params=pltpu.CompilerParams(
            collective_id=0,
            vmem_limit_bytes=40 * 1024 * 1024,
        ),
    )(x, w_mat)
